# baseline (speedup 1.0000x reference)
import functools

import jax
import jax.numpy as jnp
from jax.experimental import pallas as pl
from jax.experimental.pallas import tpu as pltpu

_INTERPRET = False


def _softplus(z):
    return jnp.log1p(jnp.exp(-jnp.abs(z))) + jnp.maximum(z, 0.0)


def _fused(x_ref, wr_ref, wa_ref, wq_ref, wk_ref, wv_ref, alog_ref, dtb_ref,
           wo_ref, y_ref, qs_scr, ks_scr, vs_scr, ld_scr, w_scr, s_scr, *,
           n_heads, topk, n_state, head_v):
    i = pl.program_id(0)
    nc = pl.num_programs(0) - 1
    f32 = jnp.float32

    @pl.when(i == 0)
    def _init():
        s_scr[...] = jnp.zeros_like(s_scr)

    @pl.when(i < nc)
    def _front():
        x = x_ref[...]
        dot = lambda a, b: jax.lax.dot_general(
            a, b, (((1,), (1,)), ((), ())), preferred_element_type=f32)

        logits = dot(x, wr_ref[...])
        a = dot(x, wa_ref[...])
        z = a + dtb_ref[...]
        ld_full = -jnp.exp(alog_ref[...]) * _softplus(z)

        q = dot(x, wq_ref[...])
        k = dot(x, wk_ref[...])
        v = dot(x, wv_ref[...])

        c = x.shape[0]
        iota_h = jax.lax.broadcasted_iota(jnp.int32, (c, n_heads), 1)
        lg = logits
        vals, idxs, sels = [], [], []
        for _ in range(topk):
            m = jnp.max(lg, axis=1, keepdims=True)
            idx = jnp.min(jnp.where(lg == m, iota_h, n_heads), axis=1,
                          keepdims=True)
            sel = iota_h == idx
            vals.append(m)
            idxs.append(idx)
            sels.append(sel)
            lg = jnp.where(sel, -1e30, lg)

        exps = [jnp.exp(val - vals[0]) for val in vals]
        denom = sum(exps)
        w_scr[i % 2] = jnp.concatenate(exps, axis=1) / denom

        ld_scr[i % 2] = jnp.concatenate(
            [jnp.sum(jnp.where(sel, ld_full, 0.0), axis=1, keepdims=True)
             for sel in sels], axis=1)

        n_bits = n_heads.bit_length() - 1
        bf16 = jnp.bfloat16
        qb = q.astype(bf16)
        kb = k.astype(bf16)
        vb = v.astype(bf16)

        idx_all = jnp.concatenate(idxs, axis=1)
        shifted = [(jax.lax.shift_right_logical(idx_all, bit) & 1) == 1
                   for bit in range(n_bits - 1, -1, -1)]
        bitsel = [[sb[:, j:j + 1] for sb in shifted] for j in range(topk)]

        def tree_select(arr, bits):
            cur = arr
            for b in bits:
                half = cur.shape[1] // 2
                cur = jnp.where(b, cur[:, half:], cur[:, :half])
            return cur

        def silu(t):
            return t * jax.nn.sigmoid(t)

        qg = jnp.concatenate([tree_select(qb, bitsel[j]) for j in range(topk)],
                             axis=1).astype(f32)
        kg = jnp.concatenate([tree_select(kb, bitsel[j]) for j in range(topk)],
                             axis=1).astype(f32)
        vg = jnp.concatenate([tree_select(vb, bitsel[j]) for j in range(topk)],
                             axis=1).astype(f32)
        qg = silu(qg)
        kg = silu(kg)
        vg = silu(vg)
        kn = topk * n_state
        g0 = jax.lax.broadcasted_iota(jnp.int32, (kn, kn), 0) // n_state
        g1 = jax.lax.broadcasted_iota(jnp.int32, (kn, kn), 1) // n_state
        bd = (g0 == g1).astype(f32)
        nq = jax.lax.dot_general(qg * qg, bd, (((1,), (0,)), ((), ())),
                                 preferred_element_type=f32)
        nk = jax.lax.dot_general(kg * kg, bd, (((1,), (0,)), ((), ())),
                                 preferred_element_type=f32)
        qs_scr[i % 2] = (qg / (jnp.sqrt(nq) + 1e-6)).astype(bf16)
        ks_scr[i % 2] = (kg / (jnp.sqrt(nk) + 1e-6)).astype(bf16)
        vs_scr[i % 2] = vg.astype(bf16)

    @pl.when(i > 0)
    def _back():
        p = (i - 1) % 2
        ld = ld_scr[p]
        w = w_scr[p]
        qs = qs_scr[p]
        ks = ks_scr[p]
        vs = vs_scr[p]
        c = ld.shape[0]
        r_iota = jax.lax.broadcasted_iota(jnp.int32, (c, c), 0)
        c_iota = jax.lax.broadcasted_iota(jnp.int32, (c, c), 1)
        mask = r_iota >= c_iota
        tri = mask.astype(f32)
        L = jax.lax.dot_general(tri, ld, (((1,), (0,)), ((), ())),
                                preferred_element_type=f32)
        LT = jax.lax.dot_general(ld, tri, (((0,), (1,)), ((), ())),
                                 preferred_element_type=f32)
        colsum = jnp.sum(ld, axis=0, keepdims=True)

        bf16 = jnp.bfloat16
        kn = topk * n_state
        rep = (jax.lax.broadcasted_iota(jnp.int32, (topk, kn), 1) // n_state
               == jax.lax.broadcasted_iota(jnp.int32, (topk, kn), 0)
               ).astype(f32)
        expand = lambda t: jax.lax.dot_general(
            t, rep, (((1,), (0,)), ((), ())), preferred_element_type=f32)
        eL_exp = expand(jnp.exp(L))
        eT_exp = expand(jnp.exp(colsum - L))
        w_exp = expand(w)
        ecs = jnp.exp(colsum)
        Qe_all = (qs.astype(f32) * eL_exp).astype(bf16)
        Ks_all = (ks.astype(f32) * eT_exp).astype(bf16)

        os = []
        for j in range(topk):
            Qj = qs[:, j * n_state:(j + 1) * n_state]
            Kj = ks[:, j * n_state:(j + 1) * n_state]
            Vj = vs[:, j * head_v:(j + 1) * head_v]
            Lj = L[:, j:j + 1]
            LTj = LT[j:j + 1, :]

            A = jax.lax.dot_general(Qj, Kj, (((1,), (1,)), ((), ())),
                                    preferred_element_type=f32)
            P = (A * jnp.exp(jnp.where(mask, Lj - LTj, -1e30))).astype(bf16)
            o = jax.lax.dot_general(P, Vj, (((1,), (0,)), ((), ())),
                                    preferred_element_type=f32)
            Sj = s_scr[j * n_state:(j + 1) * n_state, :]
            o = o + jax.lax.dot_general(
                Qe_all[:, j * n_state:(j + 1) * n_state], Sj.astype(bf16),
                (((1,), (0,)), ((), ())), preferred_element_type=f32)
            os.append(o)
            s_scr[j * n_state:(j + 1) * n_state, :] = (
                ecs[:, j:j + 1] * Sj + jax.lax.dot_general(
                    Ks_all[:, j * n_state:(j + 1) * n_state], Vj,
                    (((0,), (0,)), ((), ())), preferred_element_type=f32))

        o_all = jnp.concatenate(os, axis=1)
        fold = (jax.lax.broadcasted_iota(jnp.int32, (kn, head_v), 0) % head_v
                == jax.lax.broadcasted_iota(jnp.int32, (kn, head_v), 1)
                ).astype(f32)
        out = jax.lax.dot_general(w_exp * o_all, fold,
                                  (((1,), (0,)), ((), ())),
                                  preferred_element_type=f32)
        y_ref[...] = jax.lax.dot_general(out.astype(bf16),
                                         wo_ref[...].astype(bf16),
                                         (((1,), (1,)), ((), ())),
                                         preferred_element_type=f32)


def kernel(x, W_router, W_q, W_k, W_v, W_a, A_log, dt_bias, W_o):
    Bx, T, D = x.shape
    H = W_router.shape[0]
    HN = W_q.shape[0]
    HV = W_v.shape[0]
    n_state = HN // H
    head_v = HV // H
    topk = 8
    f32 = jnp.float32

    x2 = x.reshape(T, D)
    alog2 = A_log.reshape(1, H)
    dtb2 = dt_bias.reshape(1, H)

    C = 256 if T % 256 == 0 else T
    nc = T // C
    last = nc - 1
    full = lambda shape: pl.BlockSpec(shape, lambda i: (0, 0))

    y2 = pl.pallas_call(
        functools.partial(_fused, n_heads=H, topk=topk, n_state=n_state,
                          head_v=head_v),
        grid=(nc + 1,),
        in_specs=[pl.BlockSpec((C, D), lambda i: (jnp.minimum(i, last), 0)),
                  full((H, D)), full((H, D)), full((HN, D)), full((HN, D)),
                  full((HV, D)), full((1, H)), full((1, H)),
                  full((D, head_v))],
        out_specs=pl.BlockSpec((C, D), lambda i: (jnp.maximum(i - 1, 0), 0)),
        out_shape=jax.ShapeDtypeStruct((T, D), f32),
        scratch_shapes=[pltpu.VMEM((2, C, topk * n_state), jnp.bfloat16),
                        pltpu.VMEM((2, C, topk * n_state), jnp.bfloat16),
                        pltpu.VMEM((2, C, topk * head_v), jnp.bfloat16),
                        pltpu.VMEM((2, C, topk), f32),
                        pltpu.VMEM((2, C, topk), f32),
                        pltpu.VMEM((topk * n_state, head_v), f32)],
        compiler_params=pltpu.CompilerParams(
            dimension_semantics=("arbitrary",)),
        interpret=_INTERPRET,
    )(x2, W_router, W_a, W_q, W_k, W_v, alog2, dtb2, W_o)

    return y2.reshape(Bx, T, D)


# speedup vs baseline: 1.1281x; 1.1281x over previous
import functools

import jax
import jax.numpy as jnp
from jax.experimental import pallas as pl
from jax.experimental.pallas import tpu as pltpu

_INTERPRET = False


def _softplus(z):
    return jnp.log1p(jnp.exp(-jnp.abs(z))) + jnp.maximum(z, 0.0)


def _fused(x_ref, wr_ref, wa_ref, wq_ref, wk_ref, wv_ref, alog_ref, dtb_ref,
           wo_ref, y_ref, qs_scr, ks_scr, vs_scr, ld_scr, w_scr, s_scr, *,
           n_heads, topk, n_state, head_v):
    i = pl.program_id(0)
    nc = pl.num_programs(0) - 1
    f32 = jnp.float32

    @pl.when(i == 0)
    def _init():
        s_scr[...] = jnp.zeros_like(s_scr)

    @pl.when(i < nc)
    def _front():
        x = x_ref[...]
        dot = lambda a, b: jax.lax.dot_general(
            a, b, (((1,), (1,)), ((), ())), preferred_element_type=f32)

        logits = dot(x, wr_ref[...])
        a = dot(x, wa_ref[...])
        z = a + dtb_ref[...]
        ld_full = -jnp.exp(alog_ref[...]) * _softplus(z)

        q = dot(x, wq_ref[...])
        k = dot(x, wk_ref[...])
        v = dot(x, wv_ref[...])

        c = x.shape[0]
        iota_h = jax.lax.broadcasted_iota(jnp.int32, (c, n_heads), 1)
        lg = logits
        vals, idxs, sels = [], [], []
        for _ in range(topk):
            m = jnp.max(lg, axis=1, keepdims=True)
            idx = jnp.min(jnp.where(lg == m, iota_h, n_heads), axis=1,
                          keepdims=True)
            sel = iota_h == idx
            vals.append(m)
            idxs.append(idx)
            sels.append(sel)
            lg = jnp.where(sel, -1e30, lg)

        exps = [jnp.exp(val - vals[0]) for val in vals]
        denom = sum(exps)
        w_scr[i % 2] = jnp.concatenate(exps, axis=1) / denom

        ld_scr[i % 2] = jnp.concatenate(
            [jnp.sum(jnp.where(sel, ld_full, 0.0), axis=1, keepdims=True)
             for sel in sels], axis=1)

        n_bits = n_heads.bit_length() - 1
        bf16 = jnp.bfloat16
        qb = q.astype(bf16)
        kb = k.astype(bf16)
        vb = v.astype(bf16)

        bitsel = [[(jax.lax.shift_right_logical(idxs[j], bit) & 1) == 1
                   for bit in range(n_bits - 1, -1, -1)]
                  for j in range(topk)]

        def tree_select(arr, bits):
            cur = arr
            for b in bits:
                half = cur.shape[1] // 2
                cur = jnp.where(b, cur[:, half:], cur[:, :half])
            return cur

        def silu(t):
            return t * jax.nn.sigmoid(t)

        qg = jnp.concatenate([tree_select(qb, bitsel[j]) for j in range(topk)],
                             axis=1).astype(f32)
        kg = jnp.concatenate([tree_select(kb, bitsel[j]) for j in range(topk)],
                             axis=1).astype(f32)
        vg = jnp.concatenate([tree_select(vb, bitsel[j]) for j in range(topk)],
                             axis=1).astype(f32)
        qg = silu(qg)
        kg = silu(kg)
        vg = silu(vg)
        kn = topk * n_state
        g0 = jax.lax.broadcasted_iota(jnp.int32, (kn, kn), 0) // n_state
        g1 = jax.lax.broadcasted_iota(jnp.int32, (kn, kn), 1) // n_state
        bd = (g0 == g1).astype(f32)
        nq = jax.lax.dot_general(qg * qg, bd, (((1,), (0,)), ((), ())),
                                 preferred_element_type=f32)
        nk = jax.lax.dot_general(kg * kg, bd, (((1,), (0,)), ((), ())),
                                 preferred_element_type=f32)
        qs_scr[i % 2] = (qg / (jnp.sqrt(nq) + 1e-6)).astype(bf16)
        ks_scr[i % 2] = (kg / (jnp.sqrt(nk) + 1e-6)).astype(bf16)
        vs_scr[i % 2] = vg.astype(bf16)

    @pl.when(i > 0)
    def _back():
        p = (i - 1) % 2
        ld = ld_scr[p]
        w = w_scr[p]
        qs = qs_scr[p]
        ks = ks_scr[p]
        vs = vs_scr[p]
        c = ld.shape[0]
        r_iota = jax.lax.broadcasted_iota(jnp.int32, (c, c), 0)
        c_iota = jax.lax.broadcasted_iota(jnp.int32, (c, c), 1)
        mask = r_iota >= c_iota
        tri = mask.astype(f32)
        L = jax.lax.dot_general(tri, ld, (((1,), (0,)), ((), ())),
                                preferred_element_type=f32)
        LT = jax.lax.dot_general(ld, tri, (((0,), (1,)), ((), ())),
                                 preferred_element_type=f32)
        colsum = jnp.sum(ld, axis=0, keepdims=True)

        bf16 = jnp.bfloat16
        kn = topk * n_state
        rep = (jax.lax.broadcasted_iota(jnp.int32, (topk, kn), 1) // n_state
               == jax.lax.broadcasted_iota(jnp.int32, (topk, kn), 0)
               ).astype(f32)
        expand = lambda t: jax.lax.dot_general(
            t, rep, (((1,), (0,)), ((), ())), preferred_element_type=f32)
        eL_exp = expand(jnp.exp(L))
        eT_exp = expand(jnp.exp(colsum - L))
        w_exp = expand(w)
        ecs = jnp.exp(colsum)
        Qe_all = (qs.astype(f32) * eL_exp).astype(bf16)
        Ks_all = (ks.astype(f32) * eT_exp).astype(bf16)

        os = []
        for j in range(topk):
            Qj = qs[:, j * n_state:(j + 1) * n_state]
            Kj = ks[:, j * n_state:(j + 1) * n_state]
            Vj = vs[:, j * head_v:(j + 1) * head_v]
            Lj = L[:, j:j + 1]
            LTj = LT[j:j + 1, :]

            A = jax.lax.dot_general(Qj, Kj, (((1,), (1,)), ((), ())),
                                    preferred_element_type=f32)
            P = (A * jnp.exp(jnp.where(mask, Lj - LTj, -1e30))).astype(bf16)
            o = jax.lax.dot_general(P, Vj, (((1,), (0,)), ((), ())),
                                    preferred_element_type=f32)
            Sj = s_scr[j * n_state:(j + 1) * n_state, :]
            o = o + jax.lax.dot_general(
                Qe_all[:, j * n_state:(j + 1) * n_state], Sj.astype(bf16),
                (((1,), (0,)), ((), ())), preferred_element_type=f32)
            os.append(o)
            s_scr[j * n_state:(j + 1) * n_state, :] = (
                ecs[:, j:j + 1] * Sj + jax.lax.dot_general(
                    Ks_all[:, j * n_state:(j + 1) * n_state], Vj,
                    (((0,), (0,)), ((), ())), preferred_element_type=f32))

        o_all = jnp.concatenate(os, axis=1)
        fold = (jax.lax.broadcasted_iota(jnp.int32, (kn, head_v), 0) % head_v
                == jax.lax.broadcasted_iota(jnp.int32, (kn, head_v), 1)
                ).astype(f32)
        out = jax.lax.dot_general(w_exp * o_all, fold,
                                  (((1,), (0,)), ((), ())),
                                  preferred_element_type=f32)
        y_ref[...] = jax.lax.dot_general(out.astype(bf16),
                                         wo_ref[...].astype(bf16),
                                         (((1,), (1,)), ((), ())),
                                         preferred_element_type=f32)


def kernel(x, W_router, W_q, W_k, W_v, W_a, A_log, dt_bias, W_o):
    Bx, T, D = x.shape
    H = W_router.shape[0]
    HN = W_q.shape[0]
    HV = W_v.shape[0]
    n_state = HN // H
    head_v = HV // H
    topk = 8
    f32 = jnp.float32

    x2 = x.reshape(T, D)
    alog2 = A_log.reshape(1, H)
    dtb2 = dt_bias.reshape(1, H)

    C = 256 if T % 256 == 0 else T
    nc = T // C
    last = nc - 1
    full = lambda shape: pl.BlockSpec(shape, lambda i: (0, 0))

    y2 = pl.pallas_call(
        functools.partial(_fused, n_heads=H, topk=topk, n_state=n_state,
                          head_v=head_v),
        grid=(nc + 1,),
        in_specs=[pl.BlockSpec((C, D), lambda i: (jnp.minimum(i, last), 0)),
                  full((H, D)), full((H, D)), full((HN, D)), full((HN, D)),
                  full((HV, D)), full((1, H)), full((1, H)),
                  full((D, head_v))],
        out_specs=pl.BlockSpec((C, D), lambda i: (jnp.maximum(i - 1, 0), 0)),
        out_shape=jax.ShapeDtypeStruct((T, D), f32),
        scratch_shapes=[pltpu.VMEM((2, C, topk * n_state), jnp.bfloat16),
                        pltpu.VMEM((2, C, topk * n_state), jnp.bfloat16),
                        pltpu.VMEM((2, C, topk * head_v), jnp.bfloat16),
                        pltpu.VMEM((2, C, topk), f32),
                        pltpu.VMEM((2, C, topk), f32),
                        pltpu.VMEM((topk * n_state, head_v), f32)],
        compiler_params=pltpu.CompilerParams(
            dimension_semantics=("arbitrary",)),
        interpret=_INTERPRET,
    )(x2, W_router, W_a, W_q, W_k, W_v, alog2, dtb2, W_o)

    return y2.reshape(Bx, T, D)


# speedup vs baseline: 1.1458x; 1.0156x over previous
import functools

import jax
import jax.numpy as jnp
from jax.experimental import pallas as pl
from jax.experimental.pallas import tpu as pltpu

_INTERPRET = False


def _softplus(z):
    return jnp.log1p(jnp.exp(-jnp.abs(z))) + jnp.maximum(z, 0.0)


def _fused(x_ref, wr_ref, wa_ref, wq_ref, wk_ref, wv_ref, alog_ref, dtb_ref,
           wo_ref, y_ref, qs_scr, ks_scr, vs_scr, ld_scr, w_scr, s_scr, *,
           n_heads, topk, n_state, head_v):
    i = pl.program_id(0)
    nc = pl.num_programs(0) - 1
    f32 = jnp.float32

    @pl.when(i == 0)
    def _init():
        s_scr[...] = jnp.zeros_like(s_scr)

    @pl.when(i < nc)
    def _front():
        x = x_ref[...]
        dot = lambda a, b: jax.lax.dot_general(
            a, b, (((1,), (1,)), ((), ())), preferred_element_type=f32)

        logits = dot(x, wr_ref[...])
        a = dot(x, wa_ref[...])
        z = a + dtb_ref[...]
        ld_full = -jnp.exp(alog_ref[...]) * _softplus(z)

        q = dot(x, wq_ref[...])
        k = dot(x, wk_ref[...])
        v = dot(x, wv_ref[...])

        c = x.shape[0]
        iota_h = jax.lax.broadcasted_iota(jnp.int32, (c, n_heads), 1)
        lg = logits
        vals, idxs, sels = [], [], []
        for _ in range(topk):
            m = jnp.max(lg, axis=1, keepdims=True)
            idx = jnp.min(jnp.where(lg == m, iota_h, n_heads), axis=1,
                          keepdims=True)
            sel = iota_h == idx
            vals.append(m)
            idxs.append(idx)
            sels.append(sel)
            lg = jnp.where(sel, -1e30, lg)

        exps = [jnp.exp(val - vals[0]) for val in vals]
        denom = sum(exps)
        w_scr[i % 2] = jnp.concatenate(exps, axis=1) / denom

        ld_scr[i % 2] = jnp.concatenate(
            [jnp.sum(jnp.where(sel, ld_full, 0.0), axis=1, keepdims=True)
             for sel in sels], axis=1)

        n_bits = n_heads.bit_length() - 1
        bf16 = jnp.bfloat16
        qb = q.astype(bf16)
        kb = k.astype(bf16)
        vb = v.astype(bf16)

        bitsel = [[(jax.lax.shift_right_logical(idxs[j], bit) & 1) == 1
                   for bit in range(n_bits - 1, -1, -1)]
                  for j in range(topk)]

        def tree_select(arr, bits):
            cur = arr
            for b in bits:
                half = cur.shape[1] // 2
                cur = jnp.where(b, cur[:, half:], cur[:, :half])
            return cur

        def silu(t):
            return t * jax.nn.sigmoid(t)

        qg = jnp.concatenate([tree_select(qb, bitsel[j]) for j in range(topk)],
                             axis=1).astype(f32)
        kg = jnp.concatenate([tree_select(kb, bitsel[j]) for j in range(topk)],
                             axis=1).astype(f32)
        vg = jnp.concatenate([tree_select(vb, bitsel[j]) for j in range(topk)],
                             axis=1).astype(f32)
        qg = silu(qg)
        kg = silu(kg)
        vg = silu(vg)
        kn = topk * n_state
        g0 = jax.lax.broadcasted_iota(jnp.int32, (kn, kn), 0) // n_state
        g1 = jax.lax.broadcasted_iota(jnp.int32, (kn, kn), 1) // n_state
        bd = (g0 == g1).astype(f32)
        nq = jax.lax.dot_general(qg * qg, bd, (((1,), (0,)), ((), ())),
                                 preferred_element_type=f32)
        nk = jax.lax.dot_general(kg * kg, bd, (((1,), (0,)), ((), ())),
                                 preferred_element_type=f32)
        qs_scr[i % 2] = (qg / (jnp.sqrt(nq) + 1e-6)).astype(bf16)
        ks_scr[i % 2] = (kg / (jnp.sqrt(nk) + 1e-6)).astype(bf16)
        vs_scr[i % 2] = vg.astype(bf16)

    @pl.when(i > 0)
    def _back():
        p = (i - 1) % 2
        ld = ld_scr[p]
        w = w_scr[p]
        qs = qs_scr[p]
        ks = ks_scr[p]
        vs = vs_scr[p]
        c = ld.shape[0]
        r_iota = jax.lax.broadcasted_iota(jnp.int32, (c, c), 0)
        c_iota = jax.lax.broadcasted_iota(jnp.int32, (c, c), 1)
        mask = r_iota >= c_iota
        tri = mask.astype(f32)
        L = jax.lax.dot_general(tri, ld, (((1,), (0,)), ((), ())),
                                preferred_element_type=f32)
        LT = jax.lax.dot_general(ld, tri, (((0,), (1,)), ((), ())),
                                 preferred_element_type=f32)
        colsum = jnp.sum(ld, axis=0, keepdims=True)

        bf16 = jnp.bfloat16
        kn = topk * n_state
        rep = (jax.lax.broadcasted_iota(jnp.int32, (topk, kn), 1) // n_state
               == jax.lax.broadcasted_iota(jnp.int32, (topk, kn), 0)
               ).astype(f32)
        expand = lambda t: jax.lax.dot_general(
            t, rep, (((1,), (0,)), ((), ())), preferred_element_type=f32)
        eL_exp = expand(jnp.exp(L))
        eT_exp = expand(jnp.exp(colsum - L))
        w_exp = expand(w)
        ecs = jnp.exp(colsum)
        Qe_all = (qs.astype(f32) * eL_exp).astype(bf16)
        Ks_all = (ks.astype(f32) * eT_exp).astype(bf16)

        os = []
        for j in range(topk):
            Qj = qs[:, j * n_state:(j + 1) * n_state]
            Kj = ks[:, j * n_state:(j + 1) * n_state]
            Vj = vs[:, j * head_v:(j + 1) * head_v]
            Lj = L[:, j:j + 1]
            LTj = LT[j:j + 1, :]

            A = jax.lax.dot_general(Qj, Kj, (((1,), (1,)), ((), ())),
                                    preferred_element_type=f32)
            P = (A * jnp.exp(jnp.where(mask, Lj - LTj, -1e30))).astype(bf16)
            o = jax.lax.dot_general(P, Vj, (((1,), (0,)), ((), ())),
                                    preferred_element_type=f32)
            Sj = s_scr[j * n_state:(j + 1) * n_state, :]
            o = o + jax.lax.dot_general(
                Qe_all[:, j * n_state:(j + 1) * n_state], Sj.astype(bf16),
                (((1,), (0,)), ((), ())), preferred_element_type=f32)
            os.append(o)
            s_scr[j * n_state:(j + 1) * n_state, :] = (
                ecs[:, j:j + 1] * Sj + jax.lax.dot_general(
                    Ks_all[:, j * n_state:(j + 1) * n_state], Vj,
                    (((0,), (0,)), ((), ())), preferred_element_type=f32))

        o_all = jnp.concatenate(os, axis=1)
        fold = (jax.lax.broadcasted_iota(jnp.int32, (kn, head_v), 0) % head_v
                == jax.lax.broadcasted_iota(jnp.int32, (kn, head_v), 1)
                ).astype(f32)
        out = jax.lax.dot_general(w_exp * o_all, fold,
                                  (((1,), (0,)), ((), ())),
                                  preferred_element_type=f32)
        y_ref[...] = jax.lax.dot_general(out.astype(bf16),
                                         wo_ref[...].astype(bf16),
                                         (((1,), (1,)), ((), ())),
                                         preferred_element_type=f32)


def kernel(x, W_router, W_q, W_k, W_v, W_a, A_log, dt_bias, W_o):
    Bx, T, D = x.shape
    H = W_router.shape[0]
    HN = W_q.shape[0]
    HV = W_v.shape[0]
    n_state = HN // H
    head_v = HV // H
    topk = 8
    f32 = jnp.float32

    x2 = x.reshape(T, D)
    alog2 = A_log.reshape(1, H)
    dtb2 = dt_bias.reshape(1, H)

    C = 512 if T % 512 == 0 else T
    nc = T // C
    last = nc - 1
    full = lambda shape: pl.BlockSpec(shape, lambda i: (0, 0))

    y2 = pl.pallas_call(
        functools.partial(_fused, n_heads=H, topk=topk, n_state=n_state,
                          head_v=head_v),
        grid=(nc + 1,),
        in_specs=[pl.BlockSpec((C, D), lambda i: (jnp.minimum(i, last), 0)),
                  full((H, D)), full((H, D)), full((HN, D)), full((HN, D)),
                  full((HV, D)), full((1, H)), full((1, H)),
                  full((D, head_v))],
        out_specs=pl.BlockSpec((C, D), lambda i: (jnp.maximum(i - 1, 0), 0)),
        out_shape=jax.ShapeDtypeStruct((T, D), f32),
        scratch_shapes=[pltpu.VMEM((2, C, topk * n_state), jnp.bfloat16),
                        pltpu.VMEM((2, C, topk * n_state), jnp.bfloat16),
                        pltpu.VMEM((2, C, topk * head_v), jnp.bfloat16),
                        pltpu.VMEM((2, C, topk), f32),
                        pltpu.VMEM((2, C, topk), f32),
                        pltpu.VMEM((topk * n_state, head_v), f32)],
        compiler_params=pltpu.CompilerParams(
            dimension_semantics=("arbitrary",)),
        interpret=_INTERPRET,
    )(x2, W_router, W_a, W_q, W_k, W_v, alog2, dtb2, W_o)

    return y2.reshape(Bx, T, D)


# speedup vs baseline: 1.1491x; 1.0029x over previous
import functools

import jax
import jax.numpy as jnp
from jax.experimental import pallas as pl
from jax.experimental.pallas import tpu as pltpu

def _softplus(z):
    return jnp.log1p(jnp.exp(-jnp.abs(z))) + jnp.maximum(z, 0.0)


def _fused(x_ref, wr_ref, wa_ref, wq_ref, wk_ref, wv_ref, alog_ref, dtb_ref,
           wo_ref, y_ref, qs_scr, ks_scr, vs_scr, ld_scr, w_scr, s_scr, *,
           n_heads, topk, n_state, head_v):
    i = pl.program_id(0)
    nc = pl.num_programs(0) - 1
    f32 = jnp.float32

    @pl.when(i == 0)
    def _init():
        s_scr[...] = jnp.zeros_like(s_scr)

    @pl.when(i < nc)
    def _front():
        x = x_ref[...]
        dot = lambda a, b: jax.lax.dot_general(
            a, b, (((1,), (1,)), ((), ())), preferred_element_type=f32)

        logits = dot(x, wr_ref[...])
        a = dot(x, wa_ref[...])
        z = a + dtb_ref[...]
        ld_full = -jnp.exp(alog_ref[...]) * _softplus(z)

        q = dot(x, wq_ref[...])
        k = dot(x, wk_ref[...])
        v = dot(x, wv_ref[...])

        c = x.shape[0]
        iota_h = jax.lax.broadcasted_iota(jnp.int32, (c, n_heads), 1)
        lg = logits
        vals, idxs, sels = [], [], []
        for _ in range(topk):
            m = jnp.max(lg, axis=1, keepdims=True)
            idx = jnp.min(jnp.where(lg == m, iota_h, n_heads), axis=1,
                          keepdims=True)
            sel = iota_h == idx
            vals.append(m)
            idxs.append(idx)
            sels.append(sel)
            lg = jnp.where(sel, -1e30, lg)

        exps = [jnp.exp(val - vals[0]) for val in vals]
        denom = sum(exps)
        w_scr[i % 2] = jnp.concatenate(exps, axis=1) / denom

        ld_scr[i % 2] = jnp.concatenate(
            [jnp.sum(jnp.where(sel, ld_full, 0.0), axis=1, keepdims=True)
             for sel in sels], axis=1)

        n_bits = n_heads.bit_length() - 1
        bf16 = jnp.bfloat16
        qb = q.astype(bf16)
        kb = k.astype(bf16)
        vb = v.astype(bf16)

        bitsel = [[(jax.lax.shift_right_logical(idxs[j], bit) & 1) == 1
                   for bit in range(n_bits - 1, -1, -1)]
                  for j in range(topk)]

        def tree_select(arr, bits):
            cur = arr
            for b in bits:
                half = cur.shape[1] // 2
                cur = jnp.where(b, cur[:, half:], cur[:, :half])
            return cur

        def silu(t):
            return t * jax.nn.sigmoid(t)

        qg = jnp.concatenate([tree_select(qb, bitsel[j]) for j in range(topk)],
                             axis=1).astype(f32)
        kg = jnp.concatenate([tree_select(kb, bitsel[j]) for j in range(topk)],
                             axis=1).astype(f32)
        vg = jnp.concatenate([tree_select(vb, bitsel[j]) for j in range(topk)],
                             axis=1).astype(f32)
        qg = silu(qg)
        kg = silu(kg)
        vg = silu(vg)
        kn = topk * n_state
        g0 = jax.lax.broadcasted_iota(jnp.int32, (kn, kn), 0) // n_state
        g1 = jax.lax.broadcasted_iota(jnp.int32, (kn, kn), 1) // n_state
        bd = (g0 == g1).astype(f32)
        nq = jax.lax.dot_general(qg * qg, bd, (((1,), (0,)), ((), ())),
                                 preferred_element_type=f32)
        nk = jax.lax.dot_general(kg * kg, bd, (((1,), (0,)), ((), ())),
                                 preferred_element_type=f32)
        qs_scr[i % 2] = (qg / (jnp.sqrt(nq) + 1e-6)).astype(bf16)
        ks_scr[i % 2] = (kg / (jnp.sqrt(nk) + 1e-6)).astype(bf16)
        vs_scr[i % 2] = vg.astype(bf16)

    @pl.when(i > 0)
    def _back():
        p = (i - 1) % 2
        ld = ld_scr[p]
        w = w_scr[p]
        qs = qs_scr[p]
        ks = ks_scr[p]
        vs = vs_scr[p]
        c = ld.shape[0]
        r_iota = jax.lax.broadcasted_iota(jnp.int32, (c, c), 0)
        c_iota = jax.lax.broadcasted_iota(jnp.int32, (c, c), 1)
        mask = r_iota >= c_iota
        tri = mask.astype(f32)
        L = jax.lax.dot_general(tri, ld, (((1,), (0,)), ((), ())),
                                preferred_element_type=f32)
        LT = jax.lax.dot_general(ld, tri, (((0,), (1,)), ((), ())),
                                 preferred_element_type=f32)
        colsum = jnp.sum(ld, axis=0, keepdims=True)

        bf16 = jnp.bfloat16
        kn = topk * n_state
        rep = (jax.lax.broadcasted_iota(jnp.int32, (topk, kn), 1) // n_state
               == jax.lax.broadcasted_iota(jnp.int32, (topk, kn), 0)
               ).astype(f32)
        expand = lambda t: jax.lax.dot_general(
            t, rep, (((1,), (0,)), ((), ())), preferred_element_type=f32)
        eL_exp = expand(jnp.exp(L))
        eT_exp = expand(jnp.exp(colsum - L))
        w_exp = expand(w)
        ecs = jnp.exp(colsum)
        Qe_all = (qs.astype(f32) * eL_exp).astype(bf16)
        Ks_all = (ks.astype(f32) * eT_exp).astype(bf16)

        os = []
        for j in range(topk):
            Qj = qs[:, j * n_state:(j + 1) * n_state]
            Kj = ks[:, j * n_state:(j + 1) * n_state]
            Vj = vs[:, j * head_v:(j + 1) * head_v]
            Lj = L[:, j:j + 1]
            LTj = LT[j:j + 1, :]

            A = jax.lax.dot_general(Qj, Kj, (((1,), (1,)), ((), ())),
                                    preferred_element_type=f32)
            P = (A * jnp.exp(jnp.where(mask, Lj - LTj, -1e30))).astype(bf16)
            o = jax.lax.dot_general(P, Vj, (((1,), (0,)), ((), ())),
                                    preferred_element_type=f32)
            Sj = s_scr[j * n_state:(j + 1) * n_state, :]
            o = o + jax.lax.dot_general(
                Qe_all[:, j * n_state:(j + 1) * n_state], Sj.astype(bf16),
                (((1,), (0,)), ((), ())), preferred_element_type=f32)
            os.append(o)
            s_scr[j * n_state:(j + 1) * n_state, :] = (
                ecs[:, j:j + 1] * Sj + jax.lax.dot_general(
                    Ks_all[:, j * n_state:(j + 1) * n_state], Vj,
                    (((0,), (0,)), ((), ())), preferred_element_type=f32))

        o_all = jnp.concatenate(os, axis=1)
        fold = (jax.lax.broadcasted_iota(jnp.int32, (kn, head_v), 0) % head_v
                == jax.lax.broadcasted_iota(jnp.int32, (kn, head_v), 1)
                ).astype(f32)
        out = jax.lax.dot_general(w_exp * o_all, fold,
                                  (((1,), (0,)), ((), ())),
                                  preferred_element_type=f32)
        y_ref[...] = jax.lax.dot_general(out.astype(bf16),
                                         wo_ref[...].astype(bf16),
                                         (((1,), (1,)), ((), ())),
                                         preferred_element_type=f32)


def kernel(x, W_router, W_q, W_k, W_v, W_a, A_log, dt_bias, W_o):
    Bx, T, D = x.shape
    H = W_router.shape[0]
    HN = W_q.shape[0]
    HV = W_v.shape[0]
    n_state = HN // H
    head_v = HV // H
    topk = 8
    f32 = jnp.float32

    x2 = x.reshape(T, D)
    alog2 = A_log.reshape(1, H)
    dtb2 = dt_bias.reshape(1, H)

    C = 512 if T % 512 == 0 else T
    nc = T // C
    last = nc - 1
    full = lambda shape: pl.BlockSpec(shape, lambda i: (0, 0))

    y2 = pl.pallas_call(
        functools.partial(_fused, n_heads=H, topk=topk, n_state=n_state,
                          head_v=head_v),
        grid=(nc + 1,),
        in_specs=[pl.BlockSpec((C, D), lambda i: (jnp.minimum(i, last), 0)),
                  full((H, D)), full((H, D)), full((HN, D)), full((HN, D)),
                  full((HV, D)), full((1, H)), full((1, H)),
                  full((D, head_v))],
        out_specs=pl.BlockSpec((C, D), lambda i: (jnp.maximum(i - 1, 0), 0)),
        out_shape=jax.ShapeDtypeStruct((T, D), f32),
        scratch_shapes=[pltpu.VMEM((2, C, topk * n_state), jnp.bfloat16),
                        pltpu.VMEM((2, C, topk * n_state), jnp.bfloat16),
                        pltpu.VMEM((2, C, topk * head_v), jnp.bfloat16),
                        pltpu.VMEM((2, C, topk), f32),
                        pltpu.VMEM((2, C, topk), f32),
                        pltpu.VMEM((topk * n_state, head_v), f32)],
        compiler_params=pltpu.CompilerParams(
            dimension_semantics=("arbitrary",)),
    )(x2, W_router, W_a, W_q, W_k, W_v, alog2, dtb2, W_o)

    return y2.reshape(Bx, T, D)
